# Initial kernel scaffold; baseline (speedup 1.0000x reference)
import jax
import jax.numpy as jnp
from jax.experimental import pallas as pl


def kernel(x, Wk, Wv, Wq, Wout, w_lr, b_lr, w_mom, b_mom, w_dec, b_dec, w_gate, b_gate, Wmem0, Wmem1, Wmemout, Wvexp):
    raise NotImplementedError("write your pallas kernel here")


# speedup vs baseline: 5.7858x; 5.7858x over previous
import math

import jax
import jax.numpy as jnp
from jax.experimental import pallas as pl
from jax.experimental.pallas import tpu as pltpu

_CHUNK = 32
_NS_STEPS = 5
_NSA, _NSB, _NSC = 3.4445, -4.7750, 2.0315
_MAX_GNORM = 10.0
_F32 = jnp.float32
_INV_SQRT2 = 1.0 / math.sqrt(2.0)
_INV_SQRT2PI = 1.0 / math.sqrt(2.0 * math.pi)


def _gelu(z):
    return 0.5 * z * (1.0 + jax.lax.erf(z * _INV_SQRT2))


def _gelu_grad(z):
    cdf = 0.5 * (1.0 + jax.lax.erf(z * _INV_SQRT2))
    pdf = jnp.exp(-0.5 * z * z) * _INV_SQRT2PI
    return cdf + z * pdf


def _row_norm(t):
    return jnp.sqrt(jnp.sum(t * t, axis=1, keepdims=True))


def _proj_kernel(x_ref, wk_ref, wq_ref, wv_ref, wvexp_ref, wg_ref, bg_ref,
                 kp_ref, qp_ref, ve_ref, g_ref):
    xb = x_ref[...]

    def norm_poly(w_ref, out_ref):
        t = jax.lax.dot_general(xb, w_ref[...], (((1,), (1,)), ((), ())),
                                preferred_element_type=_F32)
        t = t / jnp.maximum(_row_norm(t), 1e-12)
        out_ref[...] = jnp.concatenate([t, t * t], axis=1) * _INV_SQRT2

    norm_poly(wk_ref, kp_ref)
    norm_poly(wq_ref, qp_ref)
    v = jax.lax.dot_general(xb, wv_ref[...], (((1,), (1,)), ((), ())),
                            preferred_element_type=_F32)
    ve_ref[...] = jnp.dot(v, wvexp_ref[...], preferred_element_type=_F32)
    gl = jax.lax.dot_general(wg_ref[...], xb, (((1,), (1,)), ((), ())),
                             preferred_element_type=_F32)
    g_ref[...] = jax.nn.sigmoid(gl + bg_ref[...])


def _ns5_into(s_ref, x_ref, a_ref, b_ref, p):
    s = s_ref[...]
    nrm = jnp.sqrt(jnp.sum(s * s))
    x_ref[...] = s / (nrm + 1e-7)
    ncol = x_ref.shape[1] // (p // 2)
    for _ in range(_NS_STEPS):
        a_ref[...] = jax.lax.dot_general(
            x_ref[...], x_ref[...], (((1,), (1,)), ((), ())),
            preferred_element_type=_F32)
        b_ref[...] = jnp.dot(a_ref[...], a_ref[...],
                             preferred_element_type=_F32)
        b_ref[...] = _NSB * a_ref[...] + _NSC * b_ref[...]
        w = p // 2
        for j in range(ncol):
            sl = slice(j * w, (j + 1) * w)
            blk = jnp.dot(b_ref[...], x_ref[:, sl], preferred_element_type=_F32)
            x_ref[:, sl] = _NSA * x_ref[:, sl] + blk


def _chunk_kernel(qk_ref, vch_ref, sc_ref, wmo_ref,
                  w0_hbm, w1_hbm, s0_hbm, s1_hbm,
                  w0n_hbm, w1n_hbm, s0n_hbm, s1n_hbm, retr_ref,
                  w0b, w1b, sb, xt, ab, bb, sem):
    nb = qk_ref.shape[0] // 2
    cp_w0 = pltpu.make_async_copy(w0_hbm, w0b, sem.at[0])
    cp_w1 = pltpu.make_async_copy(w1_hbm, w1b, sem.at[1])
    cp_s0 = pltpu.make_async_copy(s0_hbm, sb, sem.at[2])
    cp_w0.start()
    cp_w1.start()
    cp_s0.start()
    cp_w0.wait()
    cp_w1.wait()

    lrc = sc_ref[0]
    momc = sc_ref[1]
    decc = sc_ref[2]

    qk = qk_ref[...]
    z = jnp.dot(qk, w0b[...], preferred_element_type=_F32)
    h = _gelu(z)
    pr = jax.lax.dot_general(h, w1b[...], (((1,), (1,)), ((), ())),
                             preferred_element_type=_F32)
    retr_ref[...] = jnp.dot(pr[:nb], wmo_ref[...], preferred_element_type=_F32)

    kch = qk[nb:]
    hk = h[nb:]
    d = 2.0 * (pr[nb:] - vch_ref[...])
    dh = jnp.dot(d, w1b[...], preferred_element_type=_F32)
    dz = dh * _gelu_grad(z[nb:])

    inv_n = 1.0 / nb
    n0 = jnp.maximum(_row_norm(kch) * _row_norm(dz), 1e-8)
    n1 = jnp.maximum(_row_norm(hk) * _row_norm(d), 1e-8)
    wt0 = inv_n / jnp.maximum(n0 / _MAX_GNORM, 1.0)
    wt1 = inv_n / jnp.maximum(n1 / _MAX_GNORM, 1.0)

    xt[...] = jax.lax.dot_general(kch * wt0, dz, (((0,), (0,)), ((), ())),
                                  preferred_element_type=_F32)
    cp_s0.wait()
    sb[...] = momc * sb[...] - lrc * xt[...]
    cp_s0n = pltpu.make_async_copy(sb, s0n_hbm, sem.at[3])
    cp_s0n.start()
    _ns5_into(sb, xt, ab, bb, ab.shape[0])
    w0b[...] = (1.0 - decc) * w0b[...] + lrc * xt[...]
    cp_w0n = pltpu.make_async_copy(w0b, w0n_hbm, sem.at[0])
    cp_w0n.start()

    xt[...] = jax.lax.dot_general(d * wt1, hk, (((0,), (0,)), ((), ())),
                                  preferred_element_type=_F32)
    cp_s0n.wait()
    cp_s1 = pltpu.make_async_copy(s1_hbm, sb, sem.at[2])
    cp_s1.start()
    cp_s1.wait()
    sb[...] = momc * sb[...] - lrc * xt[...]
    cp_s1n = pltpu.make_async_copy(sb, s1n_hbm, sem.at[4])
    cp_s1n.start()
    _ns5_into(sb, xt, ab, bb, ab.shape[0])
    w1b[...] = (1.0 - decc) * w1b[...] + lrc * xt[...]
    cp_w1n = pltpu.make_async_copy(w1b, w1n_hbm, sem.at[1])
    cp_w1n.start()

    cp_s1n.wait()
    cp_w0n.wait()
    cp_w1n.wait()


def _out_kernel(x_ref, r_ref, wout_ref, og_ref, o_ref):
    delta = jax.lax.dot_general(r_ref[...], wout_ref[...], (((1,), (1,)), ((), ())),
                                preferred_element_type=_F32)
    o_ref[...] = x_ref[...] + delta * og_ref[...]


def kernel(x, Wk, Wv, Wq, Wout, w_lr, b_lr, w_mom, b_mom, w_dec, b_dec,
           w_gate, b_gate, Wmem0, Wmem1, Wmemout, Wvexp):
    B, S, D = x.shape
    M = Wk.shape[0]
    P, H = Wmem0.shape
    NC = S // _CHUNK
    n = B * _CHUNK
    rows = B * S
    RB = 256
    nblk = rows // RB

    xr = x.reshape(rows, D)
    wg = jnp.concatenate([w_lr, w_mom, w_dec, w_gate], axis=0)
    bg = jnp.concatenate([b_lr, b_mom, b_dec, b_gate]).reshape(4, 1)

    kp, qp, ve, gT = pl.pallas_call(
        _proj_kernel,
        grid=(nblk,),
        in_specs=[
            pl.BlockSpec((RB, D), lambda i: (i, 0)),
            pl.BlockSpec((M, D), lambda i: (0, 0)),
            pl.BlockSpec((M, D), lambda i: (0, 0)),
            pl.BlockSpec((M, D), lambda i: (0, 0)),
            pl.BlockSpec((M, P), lambda i: (0, 0)),
            pl.BlockSpec((4, D), lambda i: (0, 0)),
            pl.BlockSpec((4, 1), lambda i: (0, 0)),
        ],
        out_specs=[
            pl.BlockSpec((RB, P), lambda i: (i, 0)),
            pl.BlockSpec((RB, P), lambda i: (i, 0)),
            pl.BlockSpec((RB, P), lambda i: (i, 0)),
            pl.BlockSpec((4, RB), lambda i: (0, i)),
        ],
        out_shape=[
            jax.ShapeDtypeStruct((rows, P), _F32),
            jax.ShapeDtypeStruct((rows, P), _F32),
            jax.ShapeDtypeStruct((rows, P), _F32),
            jax.ShapeDtypeStruct((4, rows), _F32),
        ],
        compiler_params=pltpu.CompilerParams(
            dimension_semantics=("parallel",),
            vmem_limit_bytes=56 * 1024 * 1024,
        ),
        name="atlas_proj",
    )(xr, Wk, Wq, Wv, Wvexp, wg, bg)

    def to_chunks(t):
        return t.reshape(B, NC, _CHUNK, P).transpose(1, 0, 2, 3).reshape(NC, n, P)

    qk_all = jnp.concatenate([to_chunks(qp), to_chunks(kp)], axis=1)
    vc_all = to_chunks(ve)
    sc_all = gT[:3].reshape(3, B, NC, _CHUNK).mean(axis=(1, 3)).T
    og = gT[3].reshape(rows, 1)

    chunk_call = pl.pallas_call(
        _chunk_kernel,
        in_specs=[
            pl.BlockSpec(memory_space=pltpu.VMEM),
            pl.BlockSpec(memory_space=pltpu.VMEM),
            pl.BlockSpec(memory_space=pltpu.SMEM),
            pl.BlockSpec(memory_space=pltpu.VMEM),
            pl.BlockSpec(memory_space=pl.ANY),
            pl.BlockSpec(memory_space=pl.ANY),
            pl.BlockSpec(memory_space=pl.ANY),
            pl.BlockSpec(memory_space=pl.ANY),
        ],
        out_specs=[
            pl.BlockSpec(memory_space=pl.ANY),
            pl.BlockSpec(memory_space=pl.ANY),
            pl.BlockSpec(memory_space=pl.ANY),
            pl.BlockSpec(memory_space=pl.ANY),
            pl.BlockSpec(memory_space=pltpu.VMEM),
        ],
        out_shape=[
            jax.ShapeDtypeStruct((P, H), _F32),
            jax.ShapeDtypeStruct((P, H), _F32),
            jax.ShapeDtypeStruct((P, H), _F32),
            jax.ShapeDtypeStruct((P, H), _F32),
            jax.ShapeDtypeStruct((n, M), _F32),
        ],
        input_output_aliases={4: 0, 5: 1, 6: 2, 7: 3},
        scratch_shapes=[
            pltpu.VMEM((P, H), _F32),
            pltpu.VMEM((P, H), _F32),
            pltpu.VMEM((P, H), _F32),
            pltpu.VMEM((P, H), _F32),
            pltpu.VMEM((P, P), _F32),
            pltpu.VMEM((P, P), _F32),
            pltpu.SemaphoreType.DMA((6,)),
        ],
        compiler_params=pltpu.CompilerParams(
            vmem_limit_bytes=56 * 1024 * 1024,
        ),
        name="atlas_chunk",
    )

    def step(carry, xs):
        w0, w1t, s0, s1t = carry
        qki, vci, sci = xs
        w0n, w1tn, s0n, s1tn, retr = chunk_call(
            qki, vci, sci, Wmemout, w0, w1t, s0, s1t)
        return (w0n, w1tn, s0n, s1tn), retr

    init = (Wmem0, Wmem1.T, jnp.zeros((P, H), _F32), jnp.zeros((P, H), _F32))
    _, retrs = jax.lax.scan(step, init, (qk_all, vc_all, sc_all))

    retr_rows = retrs.reshape(NC, B, _CHUNK, M).transpose(1, 0, 2, 3).reshape(rows, M)

    out = pl.pallas_call(
        _out_kernel,
        grid=(nblk,),
        in_specs=[
            pl.BlockSpec((RB, D), lambda i: (i, 0)),
            pl.BlockSpec((RB, M), lambda i: (i, 0)),
            pl.BlockSpec((D, M), lambda i: (0, 0)),
            pl.BlockSpec((RB, 1), lambda i: (i, 0)),
        ],
        out_specs=pl.BlockSpec((RB, D), lambda i: (i, 0)),
        out_shape=jax.ShapeDtypeStruct((rows, D), _F32),
        compiler_params=pltpu.CompilerParams(
            dimension_semantics=("parallel",),
            vmem_limit_bytes=56 * 1024 * 1024,
        ),
        name="atlas_out",
    )(xr, retr_rows, Wout, og)

    return out.reshape(B, S, D)


# speedup vs baseline: 6.5114x; 1.1254x over previous
import math

import jax
import jax.numpy as jnp
from jax.experimental import pallas as pl
from jax.experimental.pallas import tpu as pltpu

_CHUNK = 32
_NS_STEPS = 5
_NSA, _NSB, _NSC = 3.4445, -4.7750, 2.0315
_MAX_GNORM = 10.0
_F32 = jnp.float32
_INV_SQRT2 = 1.0 / math.sqrt(2.0)
_INV_SQRT2PI = 1.0 / math.sqrt(2.0 * math.pi)


def _gelu(z):
    return 0.5 * z * (1.0 + jax.lax.erf(z * _INV_SQRT2))


def _gelu_grad(z):
    cdf = 0.5 * (1.0 + jax.lax.erf(z * _INV_SQRT2))
    pdf = jnp.exp(-0.5 * z * z) * _INV_SQRT2PI
    return cdf + z * pdf


def _row_norm(t):
    return jnp.sqrt(jnp.sum(t * t, axis=1, keepdims=True))


def _proj_kernel(x_ref, wk_ref, wq_ref, wv_ref, wvexp_ref, wg_ref, bg_ref,
                 kp_ref, qp_ref, ve_ref, g_ref):
    xb = x_ref[...]

    def norm_poly(w_ref, out_ref):
        t = jax.lax.dot_general(xb, w_ref[...], (((1,), (1,)), ((), ())),
                                preferred_element_type=_F32)
        t = t / jnp.maximum(_row_norm(t), 1e-12)
        out_ref[...] = jnp.concatenate([t, t * t], axis=1) * _INV_SQRT2

    norm_poly(wk_ref, kp_ref)
    norm_poly(wq_ref, qp_ref)
    v = jax.lax.dot_general(xb, wv_ref[...], (((1,), (1,)), ((), ())),
                            preferred_element_type=_F32)
    ve_ref[...] = jnp.dot(v, wvexp_ref[...], preferred_element_type=_F32)
    gl = jax.lax.dot_general(wg_ref[...], xb, (((1,), (1,)), ((), ())),
                             preferred_element_type=_F32)
    g_ref[...] = jax.nn.sigmoid(gl + bg_ref[...])


def _ns5_into(s_ref, x_ref, a_ref, b_ref, p):
    s = s_ref[...]
    nrm = jnp.sqrt(jnp.sum(s * s))
    x_ref[...] = s / (nrm + 1e-7)
    ncol = x_ref.shape[1] // (p // 2)
    for _ in range(_NS_STEPS):
        a_ref[...] = jax.lax.dot_general(
            x_ref[...], x_ref[...], (((1,), (1,)), ((), ())),
            preferred_element_type=_F32)
        b_ref[...] = jnp.dot(a_ref[...], a_ref[...],
                             preferred_element_type=_F32)
        b_ref[...] = _NSB * a_ref[...] + _NSC * b_ref[...]
        w = p // 2
        for j in range(ncol):
            sl = slice(j * w, (j + 1) * w)
            blk = jnp.dot(b_ref[...], x_ref[:, sl], preferred_element_type=_F32)
            x_ref[:, sl] = _NSA * x_ref[:, sl] + blk


def _scan_kernel(qk_ref, vch_ref, sc_ref, wmo_ref, w0init_hbm, w1init_hbm,
                 retr_ref,
                 w0b, w1b, s0b, s1b, xt, ab, bb, sem):
    i = pl.program_id(0)
    nb = qk_ref.shape[1] // 2

    @pl.when(i == 0)
    def _():
        cp0 = pltpu.make_async_copy(w0init_hbm, w0b, sem.at[0])
        cp1 = pltpu.make_async_copy(w1init_hbm, w1b, sem.at[1])
        cp0.start()
        cp1.start()
        s0b[...] = jnp.zeros_like(s0b)
        s1b[...] = jnp.zeros_like(s1b)
        cp0.wait()
        cp1.wait()

    lrc = sc_ref[i, 0]
    momc = sc_ref[i, 1]
    decc = sc_ref[i, 2]

    qk = qk_ref[0]
    z = jnp.dot(qk, w0b[...], preferred_element_type=_F32)
    h = _gelu(z)
    pr = jax.lax.dot_general(h, w1b[...], (((1,), (1,)), ((), ())),
                             preferred_element_type=_F32)
    retr_ref[0] = jnp.dot(pr[:nb], wmo_ref[...], preferred_element_type=_F32)

    kch = qk[nb:]
    hk = h[nb:]
    d = 2.0 * (pr[nb:] - vch_ref[0])
    dh = jnp.dot(d, w1b[...], preferred_element_type=_F32)
    dz = dh * _gelu_grad(z[nb:])

    inv_n = 1.0 / nb
    n0 = jnp.maximum(_row_norm(kch) * _row_norm(dz), 1e-8)
    n1 = jnp.maximum(_row_norm(hk) * _row_norm(d), 1e-8)
    wt0 = inv_n / jnp.maximum(n0 / _MAX_GNORM, 1.0)
    wt1 = inv_n / jnp.maximum(n1 / _MAX_GNORM, 1.0)

    xt[...] = jax.lax.dot_general(kch * wt0, dz, (((0,), (0,)), ((), ())),
                                  preferred_element_type=_F32)
    s0b[...] = momc * s0b[...] - lrc * xt[...]
    _ns5_into(s0b, xt, ab, bb, ab.shape[0])
    w0b[...] = (1.0 - decc) * w0b[...] + lrc * xt[...]

    xt[...] = jax.lax.dot_general(d * wt1, hk, (((0,), (0,)), ((), ())),
                                  preferred_element_type=_F32)
    s1b[...] = momc * s1b[...] - lrc * xt[...]
    _ns5_into(s1b, xt, ab, bb, ab.shape[0])
    w1b[...] = (1.0 - decc) * w1b[...] + lrc * xt[...]


def _out_kernel(x_ref, r_ref, wout_ref, og_ref, o_ref):
    delta = jax.lax.dot_general(r_ref[...], wout_ref[...], (((1,), (1,)), ((), ())),
                                preferred_element_type=_F32)
    o_ref[...] = x_ref[...] + delta * og_ref[...]


def kernel(x, Wk, Wv, Wq, Wout, w_lr, b_lr, w_mom, b_mom, w_dec, b_dec,
           w_gate, b_gate, Wmem0, Wmem1, Wmemout, Wvexp):
    B, S, D = x.shape
    M = Wk.shape[0]
    P, H = Wmem0.shape
    NC = S // _CHUNK
    n = B * _CHUNK
    rows = B * S
    RB = 256
    nblk = rows // RB

    xr = x.reshape(rows, D)
    wg = jnp.concatenate([w_lr, w_mom, w_dec, w_gate], axis=0)
    bg = jnp.concatenate([b_lr, b_mom, b_dec, b_gate]).reshape(4, 1)

    kp, qp, ve, gT = pl.pallas_call(
        _proj_kernel,
        grid=(nblk,),
        in_specs=[
            pl.BlockSpec((RB, D), lambda i: (i, 0)),
            pl.BlockSpec((M, D), lambda i: (0, 0)),
            pl.BlockSpec((M, D), lambda i: (0, 0)),
            pl.BlockSpec((M, D), lambda i: (0, 0)),
            pl.BlockSpec((M, P), lambda i: (0, 0)),
            pl.BlockSpec((4, D), lambda i: (0, 0)),
            pl.BlockSpec((4, 1), lambda i: (0, 0)),
        ],
        out_specs=[
            pl.BlockSpec((RB, P), lambda i: (i, 0)),
            pl.BlockSpec((RB, P), lambda i: (i, 0)),
            pl.BlockSpec((RB, P), lambda i: (i, 0)),
            pl.BlockSpec((4, RB), lambda i: (0, i)),
        ],
        out_shape=[
            jax.ShapeDtypeStruct((rows, P), _F32),
            jax.ShapeDtypeStruct((rows, P), _F32),
            jax.ShapeDtypeStruct((rows, P), _F32),
            jax.ShapeDtypeStruct((4, rows), _F32),
        ],
        compiler_params=pltpu.CompilerParams(
            dimension_semantics=("parallel",),
            vmem_limit_bytes=56 * 1024 * 1024,
        ),
        name="atlas_proj",
    )(xr, Wk, Wq, Wv, Wvexp, wg, bg)

    def to_chunks(t):
        return t.reshape(B, NC, _CHUNK, P).transpose(1, 0, 2, 3).reshape(NC, n, P)

    qk_all = jnp.concatenate([to_chunks(qp), to_chunks(kp)], axis=1)
    vc_all = to_chunks(ve)
    sc_all = gT[:3].reshape(3, B, NC, _CHUNK).mean(axis=(1, 3)).T
    og = gT[3].reshape(rows, 1)

    retrs = pl.pallas_call(
        _scan_kernel,
        grid=(NC,),
        in_specs=[
            pl.BlockSpec((1, 2 * n, P), lambda i: (i, 0, 0)),
            pl.BlockSpec((1, n, P), lambda i: (i, 0, 0)),
            pl.BlockSpec(memory_space=pltpu.SMEM),
            pl.BlockSpec((P, M), lambda i: (0, 0)),
            pl.BlockSpec(memory_space=pl.ANY),
            pl.BlockSpec(memory_space=pl.ANY),
        ],
        out_specs=pl.BlockSpec((1, n, M), lambda i: (i, 0, 0)),
        out_shape=jax.ShapeDtypeStruct((NC, n, M), _F32),
        scratch_shapes=[
            pltpu.VMEM((P, H), _F32),
            pltpu.VMEM((P, H), _F32),
            pltpu.VMEM((P, H), _F32),
            pltpu.VMEM((P, H), _F32),
            pltpu.VMEM((P, H), _F32),
            pltpu.VMEM((P, P), _F32),
            pltpu.VMEM((P, P), _F32),
            pltpu.SemaphoreType.DMA((2,)),
        ],
        compiler_params=pltpu.CompilerParams(
            dimension_semantics=("arbitrary",),
            vmem_limit_bytes=57 * 1024 * 1024,
        ),
        name="atlas_scan",
    )(qk_all, vc_all, sc_all, Wmemout, Wmem0, Wmem1.T)

    retr_rows = retrs.reshape(NC, B, _CHUNK, M).transpose(1, 0, 2, 3).reshape(rows, M)

    out = pl.pallas_call(
        _out_kernel,
        grid=(nblk,),
        in_specs=[
            pl.BlockSpec((RB, D), lambda i: (i, 0)),
            pl.BlockSpec((RB, M), lambda i: (i, 0)),
            pl.BlockSpec((D, M), lambda i: (0, 0)),
            pl.BlockSpec((RB, 1), lambda i: (i, 0)),
        ],
        out_specs=pl.BlockSpec((RB, D), lambda i: (i, 0)),
        out_shape=jax.ShapeDtypeStruct((rows, D), _F32),
        compiler_params=pltpu.CompilerParams(
            dimension_semantics=("parallel",),
            vmem_limit_bytes=56 * 1024 * 1024,
        ),
        name="atlas_out",
    )(xr, retr_rows, Wout, og)

    return out.reshape(B, S, D)


# speedup vs baseline: 7.7512x; 1.1904x over previous
import math

import jax
import jax.numpy as jnp
from jax.experimental import pallas as pl
from jax.experimental.pallas import tpu as pltpu

_CHUNK = 32
_NS_STEPS = 5
_NSA, _NSB, _NSC = 3.4445, -4.7750, 2.0315
_MAX_GNORM = 10.0
_F32 = jnp.float32
_INV_SQRT2 = 1.0 / math.sqrt(2.0)
_INV_SQRT2PI = 1.0 / math.sqrt(2.0 * math.pi)


def _gelu(z):
    return 0.5 * z * (1.0 + jax.lax.erf(z * _INV_SQRT2))


def _gelu_grad(z):
    cdf = 0.5 * (1.0 + jax.lax.erf(z * _INV_SQRT2))
    pdf = jnp.exp(-0.5 * z * z) * _INV_SQRT2PI
    return cdf + z * pdf


def _row_norm(t):
    return jnp.sqrt(jnp.sum(t * t, axis=1, keepdims=True))


def _proj_kernel(x_ref, wk_ref, wq_ref, wv_ref, wvexp_ref, wg_ref, bg_ref,
                 kp_ref, qp_ref, ve_ref, g_ref):
    xb = x_ref[...]

    def norm_poly(w_ref, out_ref):
        t = jax.lax.dot_general(xb, w_ref[...], (((1,), (1,)), ((), ())),
                                preferred_element_type=_F32)
        t = t / jnp.maximum(_row_norm(t), 1e-12)
        out_ref[...] = jnp.concatenate([t, t * t], axis=1) * _INV_SQRT2

    norm_poly(wk_ref, kp_ref)
    norm_poly(wq_ref, qp_ref)
    v = jax.lax.dot_general(xb, wv_ref[...], (((1,), (1,)), ((), ())),
                            preferred_element_type=_F32)
    ve_ref[...] = jnp.dot(v, wvexp_ref[...], preferred_element_type=_F32)
    gl = jax.lax.dot_general(wg_ref[...], xb, (((1,), (1,)), ((), ())),
                             preferred_element_type=_F32)
    g_ref[...] = jax.nn.sigmoid(gl + bg_ref[...])


def _ns5_into(s_ref, xt, a_ref, c_ref):
    pdim = a_ref.shape[0]
    s = s_ref[...]
    nrm = jnp.sqrt(jnp.sum(s * s))
    inv = 1.0 / (nrm + 1e-7)
    a_ref[...] = (inv * inv) * jax.lax.dot_general(
        s_ref[...], s_ref[...], (((1,), (1,)), ((), ())),
        preferred_element_type=_F32)
    ii = jax.lax.broadcasted_iota(jnp.int32, (pdim, pdim), 0)
    jj = jax.lax.broadcasted_iota(jnp.int32, (pdim, pdim), 1)
    eye_a = jnp.where(ii == jj, _NSA, 0.0)
    for k in range(_NS_STEPS):
        xt[:, pdim:] = jnp.dot(a_ref[...], a_ref[...],
                               preferred_element_type=_F32)
        xt[:, :pdim] = eye_a + _NSB * a_ref[...] + _NSC * xt[:, pdim:]
        if k == 0:
            c_ref[...] = xt[:, :pdim]
        else:
            xt[:, pdim:] = jnp.dot(xt[:, :pdim], c_ref[...],
                                   preferred_element_type=_F32)
            c_ref[...] = xt[:, pdim:]
        if k < _NS_STEPS - 1:
            xt[:, pdim:] = jnp.dot(xt[:, :pdim], a_ref[...],
                                   preferred_element_type=_F32)
            a_ref[...] = jnp.dot(xt[:, :pdim], xt[:, pdim:],
                                 preferred_element_type=_F32)
    xt[...] = inv * jnp.dot(c_ref[...], s_ref[...],
                            preferred_element_type=_F32)


def _scan_kernel(qk_ref, vch_ref, sc_ref, wmo_ref, w0init_hbm, w1init_hbm,
                 retr_ref,
                 w0b, w1b, s0b, s1b, xt, ab, bb, sem):
    i = pl.program_id(0)
    nb = qk_ref.shape[1] // 2

    @pl.when(i == 0)
    def _():
        cp0 = pltpu.make_async_copy(w0init_hbm, w0b, sem.at[0])
        cp1 = pltpu.make_async_copy(w1init_hbm, w1b, sem.at[1])
        cp0.start()
        cp1.start()
        s0b[...] = jnp.zeros_like(s0b)
        s1b[...] = jnp.zeros_like(s1b)
        cp0.wait()
        cp1.wait()

    lrc = sc_ref[i, 0]
    momc = sc_ref[i, 1]
    decc = sc_ref[i, 2]

    qk = qk_ref[0]
    z = jnp.dot(qk, w0b[...], preferred_element_type=_F32)
    h = _gelu(z)
    pr = jax.lax.dot_general(h, w1b[...], (((1,), (1,)), ((), ())),
                             preferred_element_type=_F32)
    retr_ref[0] = jnp.dot(pr[:nb], wmo_ref[...], preferred_element_type=_F32)

    kch = qk[nb:]
    hk = h[nb:]
    d = 2.0 * (pr[nb:] - vch_ref[0])
    dh = jnp.dot(d, w1b[...], preferred_element_type=_F32)
    dz = dh * _gelu_grad(z[nb:])

    inv_n = 1.0 / nb
    n0 = jnp.maximum(_row_norm(kch) * _row_norm(dz), 1e-8)
    n1 = jnp.maximum(_row_norm(hk) * _row_norm(d), 1e-8)
    wt0 = inv_n / jnp.maximum(n0 / _MAX_GNORM, 1.0)
    wt1 = inv_n / jnp.maximum(n1 / _MAX_GNORM, 1.0)

    xt[...] = jax.lax.dot_general(kch * wt0, dz, (((0,), (0,)), ((), ())),
                                  preferred_element_type=_F32)
    s0b[...] = momc * s0b[...] - lrc * xt[...]
    _ns5_into(s0b, xt, ab, bb)
    w0b[...] = (1.0 - decc) * w0b[...] + lrc * xt[...]

    xt[...] = jax.lax.dot_general(d * wt1, hk, (((0,), (0,)), ((), ())),
                                  preferred_element_type=_F32)
    s1b[...] = momc * s1b[...] - lrc * xt[...]
    _ns5_into(s1b, xt, ab, bb)
    w1b[...] = (1.0 - decc) * w1b[...] + lrc * xt[...]


def _out_kernel(x_ref, r_ref, wout_ref, og_ref, o_ref):
    delta = jax.lax.dot_general(r_ref[...], wout_ref[...], (((1,), (1,)), ((), ())),
                                preferred_element_type=_F32)
    o_ref[...] = x_ref[...] + delta * og_ref[...]


def kernel(x, Wk, Wv, Wq, Wout, w_lr, b_lr, w_mom, b_mom, w_dec, b_dec,
           w_gate, b_gate, Wmem0, Wmem1, Wmemout, Wvexp):
    B, S, D = x.shape
    M = Wk.shape[0]
    P, H = Wmem0.shape
    NC = S // _CHUNK
    n = B * _CHUNK
    rows = B * S
    RB = 256
    nblk = rows // RB

    xr = x.reshape(rows, D)
    wg = jnp.concatenate([w_lr, w_mom, w_dec, w_gate], axis=0)
    bg = jnp.concatenate([b_lr, b_mom, b_dec, b_gate]).reshape(4, 1)

    kp, qp, ve, gT = pl.pallas_call(
        _proj_kernel,
        grid=(nblk,),
        in_specs=[
            pl.BlockSpec((RB, D), lambda i: (i, 0)),
            pl.BlockSpec((M, D), lambda i: (0, 0)),
            pl.BlockSpec((M, D), lambda i: (0, 0)),
            pl.BlockSpec((M, D), lambda i: (0, 0)),
            pl.BlockSpec((M, P), lambda i: (0, 0)),
            pl.BlockSpec((4, D), lambda i: (0, 0)),
            pl.BlockSpec((4, 1), lambda i: (0, 0)),
        ],
        out_specs=[
            pl.BlockSpec((RB, P), lambda i: (i, 0)),
            pl.BlockSpec((RB, P), lambda i: (i, 0)),
            pl.BlockSpec((RB, P), lambda i: (i, 0)),
            pl.BlockSpec((4, RB), lambda i: (0, i)),
        ],
        out_shape=[
            jax.ShapeDtypeStruct((rows, P), _F32),
            jax.ShapeDtypeStruct((rows, P), _F32),
            jax.ShapeDtypeStruct((rows, P), _F32),
            jax.ShapeDtypeStruct((4, rows), _F32),
        ],
        compiler_params=pltpu.CompilerParams(
            dimension_semantics=("parallel",),
            vmem_limit_bytes=56 * 1024 * 1024,
        ),
        name="atlas_proj",
    )(xr, Wk, Wq, Wv, Wvexp, wg, bg)

    def to_chunks(t):
        return t.reshape(B, NC, _CHUNK, P).transpose(1, 0, 2, 3).reshape(NC, n, P)

    qk_all = jnp.concatenate([to_chunks(qp), to_chunks(kp)], axis=1)
    vc_all = to_chunks(ve)
    sc_all = gT[:3].reshape(3, B, NC, _CHUNK).mean(axis=(1, 3)).T
    og = gT[3].reshape(rows, 1)

    retrs = pl.pallas_call(
        _scan_kernel,
        grid=(NC,),
        in_specs=[
            pl.BlockSpec((1, 2 * n, P), lambda i: (i, 0, 0)),
            pl.BlockSpec((1, n, P), lambda i: (i, 0, 0)),
            pl.BlockSpec(memory_space=pltpu.SMEM),
            pl.BlockSpec((P, M), lambda i: (0, 0)),
            pl.BlockSpec(memory_space=pl.ANY),
            pl.BlockSpec(memory_space=pl.ANY),
        ],
        out_specs=pl.BlockSpec((1, n, M), lambda i: (i, 0, 0)),
        out_shape=jax.ShapeDtypeStruct((NC, n, M), _F32),
        scratch_shapes=[
            pltpu.VMEM((P, H), _F32),
            pltpu.VMEM((P, H), _F32),
            pltpu.VMEM((P, H), _F32),
            pltpu.VMEM((P, H), _F32),
            pltpu.VMEM((P, H), _F32),
            pltpu.VMEM((P, P), _F32),
            pltpu.VMEM((P, P), _F32),
            pltpu.SemaphoreType.DMA((2,)),
        ],
        compiler_params=pltpu.CompilerParams(
            dimension_semantics=("arbitrary",),
            vmem_limit_bytes=57 * 1024 * 1024,
        ),
        name="atlas_scan",
    )(qk_all, vc_all, sc_all, Wmemout, Wmem0, Wmem1.T)

    retr_rows = retrs.reshape(NC, B, _CHUNK, M).transpose(1, 0, 2, 3).reshape(rows, M)

    out = pl.pallas_call(
        _out_kernel,
        grid=(nblk,),
        in_specs=[
            pl.BlockSpec((RB, D), lambda i: (i, 0)),
            pl.BlockSpec((RB, M), lambda i: (i, 0)),
            pl.BlockSpec((D, M), lambda i: (0, 0)),
            pl.BlockSpec((RB, 1), lambda i: (i, 0)),
        ],
        out_specs=pl.BlockSpec((RB, D), lambda i: (i, 0)),
        out_shape=jax.ShapeDtypeStruct((rows, D), _F32),
        compiler_params=pltpu.CompilerParams(
            dimension_semantics=("parallel",),
            vmem_limit_bytes=56 * 1024 * 1024,
        ),
        name="atlas_out",
    )(xr, retr_rows, Wout, og)

    return out.reshape(B, S, D)


# speedup vs baseline: 7.8426x; 1.0118x over previous
import math

import jax
import jax.numpy as jnp
from jax.experimental import pallas as pl
from jax.experimental.pallas import tpu as pltpu

_CHUNK = 32
_NS_STEPS = 5
_NSA, _NSB, _NSC = 3.4445, -4.7750, 2.0315
_MAX_GNORM = 10.0
_F32 = jnp.float32
_INV_SQRT2 = 1.0 / math.sqrt(2.0)
_INV_SQRT2PI = 1.0 / math.sqrt(2.0 * math.pi)


def _gelu(z):
    return 0.5 * z * (1.0 + jax.lax.erf(z * _INV_SQRT2))


def _gelu_grad(z):
    cdf = 0.5 * (1.0 + jax.lax.erf(z * _INV_SQRT2))
    pdf = jnp.exp(-0.5 * z * z) * _INV_SQRT2PI
    return cdf + z * pdf


def _row_norm(t):
    return jnp.sqrt(jnp.sum(t * t, axis=1, keepdims=True))


def _proj_kernel(x_ref, wk_ref, wq_ref, wv_ref, wvexp_ref, wg_ref, bg_ref,
                 kp_ref, qp_ref, ve_ref, g_ref):
    xb = x_ref[...]
    cb, ch, pdim = kp_ref.shape

    def norm_poly(w_ref, out_ref):
        t = jax.lax.dot_general(xb, w_ref[...], (((1,), (1,)), ((), ())),
                                preferred_element_type=_F32)
        t = t / jnp.maximum(_row_norm(t), 1e-12)
        out_ref[...] = (jnp.concatenate([t, t * t], axis=1)
                        * _INV_SQRT2).reshape(cb, ch, pdim)

    norm_poly(wk_ref, kp_ref)
    norm_poly(wq_ref, qp_ref)
    v = jax.lax.dot_general(xb, wv_ref[...], (((1,), (1,)), ((), ())),
                            preferred_element_type=_F32)
    ve_ref[...] = jnp.dot(v, wvexp_ref[...],
                          preferred_element_type=_F32).reshape(cb, ch, pdim)
    gl = jax.lax.dot_general(wg_ref[...], xb, (((1,), (1,)), ((), ())),
                             preferred_element_type=_F32)
    g_ref[...] = jax.nn.sigmoid(gl + bg_ref[...])


def _ns5_into(s_ref, xt, a_ref, c_ref):
    pdim = a_ref.shape[0]
    s = s_ref[...]
    nrm = jnp.sqrt(jnp.sum(s * s))
    inv = 1.0 / (nrm + 1e-7)
    a_ref[...] = (inv * inv) * jax.lax.dot_general(
        s_ref[...], s_ref[...], (((1,), (1,)), ((), ())),
        preferred_element_type=_F32)
    ii = jax.lax.broadcasted_iota(jnp.int32, (pdim, pdim), 0)
    jj = jax.lax.broadcasted_iota(jnp.int32, (pdim, pdim), 1)
    eye_a = jnp.where(ii == jj, _NSA, 0.0)
    for k in range(_NS_STEPS):
        xt[:, pdim:] = jnp.dot(a_ref[...], a_ref[...],
                               preferred_element_type=_F32)
        xt[:, :pdim] = eye_a + _NSB * a_ref[...] + _NSC * xt[:, pdim:]
        if k == 0:
            c_ref[...] = xt[:, :pdim]
        else:
            xt[:, pdim:] = jnp.dot(xt[:, :pdim], c_ref[...],
                                   preferred_element_type=_F32)
            c_ref[...] = xt[:, pdim:]
        if k < _NS_STEPS - 1:
            xt[:, pdim:] = jnp.dot(xt[:, :pdim], a_ref[...],
                                   preferred_element_type=_F32)
            a_ref[...] = jnp.dot(xt[:, :pdim], xt[:, pdim:],
                                 preferred_element_type=_F32)
    xt[...] = inv * jnp.dot(c_ref[...], s_ref[...],
                            preferred_element_type=_F32)


def _scan_kernel(qch_ref, kch_ref, vch_ref, sc_ref, wmo_ref, w0init_hbm,
                 w1init_hbm, retr_ref,
                 w0b, w1b, s0b, s1b, xt, ab, bb, sem):
    i = pl.program_id(0)
    nb = kch_ref.shape[1]

    @pl.when(i == 0)
    def _():
        cp0 = pltpu.make_async_copy(w0init_hbm, w0b, sem.at[0])
        cp1 = pltpu.make_async_copy(w1init_hbm, w1b, sem.at[1])
        cp0.start()
        cp1.start()
        s0b[...] = jnp.zeros_like(s0b)
        s1b[...] = jnp.zeros_like(s1b)
        cp0.wait()
        cp1.wait()

    lrc = sc_ref[i, 0]
    momc = sc_ref[i, 1]
    decc = sc_ref[i, 2]

    kch = kch_ref[0]
    qk = jnp.concatenate([qch_ref[0], kch], axis=0)
    z = jnp.dot(qk, w0b[...], preferred_element_type=_F32)
    h = _gelu(z)
    pr = jax.lax.dot_general(h, w1b[...], (((1,), (1,)), ((), ())),
                             preferred_element_type=_F32)
    retr_ref[...] = jnp.dot(pr[:nb], wmo_ref[...],
                            preferred_element_type=_F32).reshape(retr_ref.shape)
    hk = h[nb:]
    d = 2.0 * (pr[nb:] - vch_ref[0])
    dh = jnp.dot(d, w1b[...], preferred_element_type=_F32)
    dz = dh * _gelu_grad(z[nb:])

    inv_n = 1.0 / nb
    n0 = jnp.maximum(_row_norm(kch) * _row_norm(dz), 1e-8)
    n1 = jnp.maximum(_row_norm(hk) * _row_norm(d), 1e-8)
    wt0 = inv_n / jnp.maximum(n0 / _MAX_GNORM, 1.0)
    wt1 = inv_n / jnp.maximum(n1 / _MAX_GNORM, 1.0)

    xt[...] = jax.lax.dot_general(kch * wt0, dz, (((0,), (0,)), ((), ())),
                                  preferred_element_type=_F32)
    s0b[...] = momc * s0b[...] - lrc * xt[...]
    _ns5_into(s0b, xt, ab, bb)
    w0b[...] = (1.0 - decc) * w0b[...] + lrc * xt[...]

    xt[...] = jax.lax.dot_general(d * wt1, hk, (((0,), (0,)), ((), ())),
                                  preferred_element_type=_F32)
    s1b[...] = momc * s1b[...] - lrc * xt[...]
    _ns5_into(s1b, xt, ab, bb)
    w1b[...] = (1.0 - decc) * w1b[...] + lrc * xt[...]


def _out_kernel(x_ref, r_ref, wout_ref, og_ref, o_ref):
    delta = jax.lax.dot_general(r_ref[...], wout_ref[...], (((1,), (1,)), ((), ())),
                                preferred_element_type=_F32)
    o_ref[...] = x_ref[...] + delta * og_ref[...]


def kernel(x, Wk, Wv, Wq, Wout, w_lr, b_lr, w_mom, b_mom, w_dec, b_dec,
           w_gate, b_gate, Wmem0, Wmem1, Wmemout, Wvexp):
    B, S, D = x.shape
    M = Wk.shape[0]
    P, H = Wmem0.shape
    NC = S // _CHUNK
    n = B * _CHUNK
    rows = B * S
    RB = min(256, S)
    nblk = rows // RB
    CPB = RB // _CHUNK

    xr = x.reshape(rows, D)
    wg = jnp.concatenate([w_lr, w_mom, w_dec, w_gate], axis=0)
    bg = jnp.concatenate([b_lr, b_mom, b_dec, b_gate]).reshape(4, 1)

    kp, qp, ve, gT = pl.pallas_call(
        _proj_kernel,
        grid=(NC // CPB, B),
        in_specs=[
            pl.BlockSpec((RB, D), lambda cb, b: (b * (NC // CPB) + cb, 0)),
            pl.BlockSpec((M, D), lambda cb, b: (0, 0)),
            pl.BlockSpec((M, D), lambda cb, b: (0, 0)),
            pl.BlockSpec((M, D), lambda cb, b: (0, 0)),
            pl.BlockSpec((M, P), lambda cb, b: (0, 0)),
            pl.BlockSpec((4, D), lambda cb, b: (0, 0)),
            pl.BlockSpec((4, 1), lambda cb, b: (0, 0)),
        ],
        out_specs=[
            pl.BlockSpec((CPB, _CHUNK, P), lambda cb, b: (cb, b, 0)),
            pl.BlockSpec((CPB, _CHUNK, P), lambda cb, b: (cb, b, 0)),
            pl.BlockSpec((CPB, _CHUNK, P), lambda cb, b: (cb, b, 0)),
            pl.BlockSpec((4, RB), lambda cb, b: (0, b * (NC // CPB) + cb)),
        ],
        out_shape=[
            jax.ShapeDtypeStruct((NC, n, P), _F32),
            jax.ShapeDtypeStruct((NC, n, P), _F32),
            jax.ShapeDtypeStruct((NC, n, P), _F32),
            jax.ShapeDtypeStruct((4, rows), _F32),
        ],
        compiler_params=pltpu.CompilerParams(
            dimension_semantics=("arbitrary", "arbitrary"),
            vmem_limit_bytes=56 * 1024 * 1024,
        ),
        name="atlas_proj",
    )(xr, Wk, Wq, Wv, Wvexp, wg, bg)

    sc_all = gT[:3].reshape(3, B, NC, _CHUNK).mean(axis=(1, 3)).T
    og = gT[3].reshape(rows, 1)

    retrs = pl.pallas_call(
        _scan_kernel,
        grid=(NC,),
        in_specs=[
            pl.BlockSpec((1, n, P), lambda i: (i, 0, 0)),
            pl.BlockSpec((1, n, P), lambda i: (i, 0, 0)),
            pl.BlockSpec((1, n, P), lambda i: (i, 0, 0)),
            pl.BlockSpec(memory_space=pltpu.SMEM),
            pl.BlockSpec((P, M), lambda i: (0, 0)),
            pl.BlockSpec(memory_space=pl.ANY),
            pl.BlockSpec(memory_space=pl.ANY),
        ],
        out_specs=pl.BlockSpec((B, 1, _CHUNK, M), lambda i: (0, i, 0, 0)),
        out_shape=jax.ShapeDtypeStruct((B, NC, _CHUNK, M), _F32),
        scratch_shapes=[
            pltpu.VMEM((P, H), _F32),
            pltpu.VMEM((P, H), _F32),
            pltpu.VMEM((P, H), _F32),
            pltpu.VMEM((P, H), _F32),
            pltpu.VMEM((P, H), _F32),
            pltpu.VMEM((P, P), _F32),
            pltpu.VMEM((P, P), _F32),
            pltpu.SemaphoreType.DMA((2,)),
        ],
        compiler_params=pltpu.CompilerParams(
            dimension_semantics=("arbitrary",),
            vmem_limit_bytes=57 * 1024 * 1024,
        ),
        name="atlas_scan",
    )(qp, kp, ve, sc_all, Wmemout, Wmem0, Wmem1.T)

    retr_rows = retrs.reshape(rows, M)

    out = pl.pallas_call(
        _out_kernel,
        grid=(nblk,),
        in_specs=[
            pl.BlockSpec((RB, D), lambda i: (i, 0)),
            pl.BlockSpec((RB, M), lambda i: (i, 0)),
            pl.BlockSpec((D, M), lambda i: (0, 0)),
            pl.BlockSpec((RB, 1), lambda i: (i, 0)),
        ],
        out_specs=pl.BlockSpec((RB, D), lambda i: (i, 0)),
        out_shape=jax.ShapeDtypeStruct((rows, D), _F32),
        compiler_params=pltpu.CompilerParams(
            dimension_semantics=("parallel",),
            vmem_limit_bytes=56 * 1024 * 1024,
        ),
        name="atlas_out",
    )(xr, retr_rows, Wout, og)

    return out.reshape(B, S, D)
